# baseline (speedup 1.0000x reference)
import functools

import jax
import jax.numpy as jnp
from jax import lax
from jax.experimental import pallas as pl
from jax.experimental.pallas import tpu as pltpu
from jax.experimental.pallas import tpu_sc as plsc

D = 64
K = 1024
BETA = 0.25

ROWS = 16 * 1024
BLOCK_ROWS = 512
NUM_BLOCKS = ROWS // BLOCK_ROWS

NUM_WORKERS = 32
BPW = ROWS // NUM_WORKERS
CHUNK = 128
NCHUNKS = BPW // CHUNK


def _assign_body(z_ref, e_ref, zn_ref, en_ref, idx_ref, loss_ref):
    i = pl.program_id(0)
    z = z_ref[...]
    e = e_ref[...]
    s = lax.dot_general(z, e, (((1,), (1,)), ((), ())),
                        preferred_element_type=jnp.float32)
    zn = zn_ref[0, 0, :]
    en = en_ref[0, :]
    d = (zn[:, None] + en[None, :]) - 2.0 * s
    row_min = jnp.min(d, axis=1)
    ids = lax.broadcasted_iota(jnp.int32, d.shape, 1)
    idx = jnp.min(jnp.where(d == row_min[:, None], ids, K), axis=1)
    idx_ref[0, 0, :] = idx
    partial = jnp.sum(row_min)

    @pl.when(i == 0)
    def _init():
        loss_ref[0, 0] = partial

    @pl.when(i != 0)
    def _acc():
        loss_ref[0, 0] += partial


def _assign(z_flat, embeddings, z_norm, e_norm):
    return pl.pallas_call(
        _assign_body,
        grid=(NUM_BLOCKS,),
        in_specs=[
            pl.BlockSpec((BLOCK_ROWS, D), lambda i: (i, 0)),
            pl.BlockSpec((K, D), lambda i: (0, 0)),
            pl.BlockSpec((1, 1, BLOCK_ROWS), lambda i: (i, 0, 0)),
            pl.BlockSpec((1, K), lambda i: (0, 0)),
        ],
        out_specs=[
            pl.BlockSpec((1, 1, BLOCK_ROWS), lambda i: (i, 0, 0)),
            pl.BlockSpec((1, 1), lambda i: (0, 0), memory_space=pltpu.SMEM),
        ],
        out_shape=[
            jax.ShapeDtypeStruct((NUM_BLOCKS, 1, BLOCK_ROWS), jnp.int32),
            jax.ShapeDtypeStruct((1, 1), jnp.float32),
        ],
    )(z_flat, embeddings, z_norm, e_norm)


def _gather_body(emb_hbm, idx_hbm, out_hbm, idx_v, rows_v, sem):
    wid = lax.axis_index("s") * 2 + lax.axis_index("c")
    base = wid * BPW
    pltpu.sync_copy(idx_hbm.at[wid], idx_v)
    copies = [
        pltpu.async_copy(
            emb_hbm.at[idx_v.at[j]],
            rows_v.at[pl.ds(j * CHUNK, CHUNK)],
            sem,
        )
        for j in range(NCHUNKS)
    ]
    for c in copies:
        c.wait()
    pltpu.sync_copy(rows_v, out_hbm.at[pl.ds(base, BPW)])


@functools.cache
def _gather():
    mesh = plsc.VectorSubcoreMesh(core_axis_name="c", subcore_axis_name="s")
    return pl.kernel(
        _gather_body,
        out_type=jax.ShapeDtypeStruct((ROWS, D), jnp.float32),
        mesh=mesh,
        scratch_types=[
            pltpu.VMEM((NCHUNKS, CHUNK), jnp.int32),
            pltpu.VMEM((BPW, D), jnp.float32),
            pltpu.SemaphoreType.DMA,
        ],
        compiler_params=pltpu.CompilerParams(use_tc_tiling_on_sc=False),
    )


def kernel(z_e, embeddings):
    z_flat = z_e.reshape(ROWS, D)
    z_norm = jnp.sum(z_flat ** 2, axis=1).reshape(NUM_BLOCKS, 1, BLOCK_ROWS)
    e_norm = jnp.sum(embeddings ** 2, axis=1).reshape(1, K)
    idx3, loss_sum = _assign(z_flat, embeddings, z_norm, e_norm)
    idx = idx3.reshape(ROWS)
    z_q = jnp.take(embeddings, idx, axis=0)
    vq_loss = loss_sum[0, 0] * ((1.0 + BETA) / float(ROWS * D))
    return z_q.reshape(z_e.shape), vq_loss


# speedup vs baseline: 1.4991x; 1.4991x over previous
import functools

import jax
import jax.numpy as jnp
from jax import lax
from jax.experimental import pallas as pl
from jax.experimental.pallas import tpu as pltpu
from jax.experimental.pallas import tpu_sc as plsc

D = 64
K = 1024
BETA = 0.25

ROWS = 16 * 1024
BLOCK_ROWS = 512
NUM_BLOCKS = ROWS // BLOCK_ROWS

NUM_WORKERS = 32
BPW = ROWS // NUM_WORKERS
CHUNK = 128
NCHUNKS = BPW // CHUNK


def _assign_body(z_ref, e_ref, zn_ref, en_ref, idx_ref, loss_ref, zq_ref):
    i = pl.program_id(0)
    z = z_ref[...]
    e = e_ref[...]
    s = lax.dot_general(z, e, (((1,), (1,)), ((), ())),
                        preferred_element_type=jnp.float32)
    zn = zn_ref[0, 0, :]
    en = en_ref[0, :]
    d = (zn[:, None] + en[None, :]) - 2.0 * s
    row_min = jnp.min(d, axis=1)
    ids = lax.broadcasted_iota(jnp.int32, d.shape, 1)
    idx = jnp.min(jnp.where(d == row_min[:, None], ids, K), axis=1)
    idx_ref[0, 0, :] = idx
    one_hot = (ids == idx[:, None]).astype(jnp.float32)
    zq_ref[...] = lax.dot_general(one_hot, e_ref[...], (((1,), (0,)), ((), ())),
                                  preferred_element_type=jnp.float32)
    partial = jnp.sum(row_min)

    @pl.when(i == 0)
    def _init():
        loss_ref[0, 0] = partial

    @pl.when(i != 0)
    def _acc():
        loss_ref[0, 0] += partial


def _assign(z_flat, embeddings, z_norm, e_norm):
    return pl.pallas_call(
        _assign_body,
        grid=(NUM_BLOCKS,),
        in_specs=[
            pl.BlockSpec((BLOCK_ROWS, D), lambda i: (i, 0)),
            pl.BlockSpec((K, D), lambda i: (0, 0)),
            pl.BlockSpec((1, 1, BLOCK_ROWS), lambda i: (i, 0, 0)),
            pl.BlockSpec((1, K), lambda i: (0, 0)),
        ],
        out_specs=[
            pl.BlockSpec((1, 1, BLOCK_ROWS), lambda i: (i, 0, 0)),
            pl.BlockSpec((1, 1), lambda i: (0, 0), memory_space=pltpu.SMEM),
            pl.BlockSpec((BLOCK_ROWS, D), lambda i: (i, 0)),
        ],
        out_shape=[
            jax.ShapeDtypeStruct((NUM_BLOCKS, 1, BLOCK_ROWS), jnp.int32),
            jax.ShapeDtypeStruct((1, 1), jnp.float32),
            jax.ShapeDtypeStruct((ROWS, D), jnp.float32),
        ],
    )(z_flat, embeddings, z_norm, e_norm)


def _gather_body(emb_hbm, idx_hbm, out_hbm, idx_v, rows_v, sem):
    wid = lax.axis_index("s") * 2 + lax.axis_index("c")
    base = wid * BPW
    pltpu.sync_copy(idx_hbm.at[wid], idx_v)
    copies = [
        pltpu.async_copy(
            emb_hbm.at[idx_v.at[j]],
            rows_v.at[pl.ds(j * CHUNK, CHUNK)],
            sem,
        )
        for j in range(NCHUNKS)
    ]
    for c in copies:
        c.wait()
    pltpu.sync_copy(rows_v, out_hbm.at[pl.ds(base, BPW)])


@functools.cache
def _gather():
    mesh = plsc.VectorSubcoreMesh(core_axis_name="c", subcore_axis_name="s")
    return pl.kernel(
        _gather_body,
        out_type=jax.ShapeDtypeStruct((ROWS, D), jnp.float32),
        mesh=mesh,
        scratch_types=[
            pltpu.VMEM((NCHUNKS, CHUNK), jnp.int32),
            pltpu.VMEM((BPW, D), jnp.float32),
            pltpu.SemaphoreType.DMA,
        ],
        compiler_params=pltpu.CompilerParams(use_tc_tiling_on_sc=False),
    )


def kernel(z_e, embeddings):
    z_flat = z_e.reshape(ROWS, D)
    z_norm = jnp.sum(z_flat ** 2, axis=1).reshape(NUM_BLOCKS, 1, BLOCK_ROWS)
    e_norm = jnp.sum(embeddings ** 2, axis=1).reshape(1, K)
    idx3, loss_sum, z_q = _assign(z_flat, embeddings, z_norm, e_norm)
    vq_loss = loss_sum[0, 0] * ((1.0 + BETA) / float(ROWS * D))
    return z_q.reshape(z_e.shape), vq_loss
